# baseline (speedup 1.0000x reference)
import functools

import jax
import jax.numpy as jnp
from jax import lax
from jax.experimental import pallas as pl
from jax.experimental.pallas import tpu as pltpu
from jax.experimental.pallas import tpu_sc as plsc

HIDDEN = 128
EPS = 1e-12
LANES = 16
SEGS = HIDDEN // LANES
CHUNK = 128
UNROLL = 4
RING = 4

_GATHER_DNUMS = lax.GatherDimensionNumbers(
    offset_dims=(), collapsed_slice_dims=(0,), start_index_map=(0,))


def _lane_shuffle(x, idx):
    return lax.gather(x, idx.reshape(LANES, 1), _GATHER_DNUMS, (1,),
                      mode=lax.GatherScatterMode.PROMISE_IN_BOUNDS)


def _hsum_all_lanes(x):
    lane = lax.iota(jnp.int32, LANES)
    for k in (8, 4, 2, 1):
        x = x + _lane_shuffle(x, lane ^ k)
    return x


def _rsqrt_vec(v):
    i = lax.bitcast_convert_type(v, jnp.int32)
    i = jnp.int32(0x5F3759DF) - lax.shift_right_logical(i, 1)
    y = lax.bitcast_convert_type(i, jnp.float32)
    return y * (jnp.float32(1.5) - (v * jnp.float32(0.5)) * y * y)


def _normalize_row(in_ref, out_ref, r):
    segs = [in_ref[r, pl.ds(LANES * j, LANES)] for j in range(SEGS)]
    s = segs[0]
    s2 = segs[0] * segs[0]
    for j in range(1, SEGS):
        s = s + segs[j]
        s2 = s2 + segs[j] * segs[j]
    mv = _hsum_all_lanes(s) * jnp.float32(1.0 / HIDDEN)
    rv = (_hsum_all_lanes(s2) * jnp.float32(1.0 / HIDDEN)
          - mv * mv + jnp.float32(EPS))
    y = _rsqrt_vec(rv)
    for j in range(SEGS):
        out_ref[r, pl.ds(LANES * j, LANES)] = (segs[j] - mv) * y


def _make_mesh():
    return plsc.VectorSubcoreMesh(core_axis_name="c", subcore_axis_name="s")


def _num_workers():
    info = plsc.get_sparse_core_info()
    return info.num_cores * info.num_subcores, info.num_cores


@functools.partial(jax.jit, static_argnames=("vocab",))
def _sc_normalize_table(table, vocab):
    nw, nc = _num_workers()
    n_chunks = 25

    @functools.partial(
        pl.kernel,
        out_type=jax.ShapeDtypeStruct((vocab, HIDDEN), jnp.float32),
        mesh=_make_mesh(),
        scratch_types=[
            pltpu.VMEM((CHUNK, HIDDEN), jnp.float32),
            pltpu.VMEM((CHUNK, HIDDEN), jnp.float32),
            pltpu.VMEM((CHUNK, HIDDEN), jnp.float32),
            pltpu.VMEM((CHUNK, HIDDEN), jnp.float32),
            pltpu.SemaphoreType.DMA,
            pltpu.SemaphoreType.DMA,
            pltpu.SemaphoreType.DMA,
            pltpu.SemaphoreType.DMA,
        ],
    )
    def ka(table_hbm, normed_hbm, in0, in1, out0, out1,
           gsem0, gsem1, osem0, osem1):
        wid = lax.axis_index("s") * nc + lax.axis_index("c")
        bufs = ((in0, out0, gsem0, osem0), (in1, out1, gsem1, osem1))

        def start(c):
            return jnp.minimum((wid * n_chunks + c) * CHUNK, vocab - CHUNK)

        def src(c):
            return table_hbm.at[pl.ds(start(c), CHUNK)]

        def dst(c):
            return normed_hbm.at[pl.ds(start(c), CHUNK)]

        def do_chunk(i, c, inb, outb, gsem, osem, last):
            pltpu.make_async_copy(src(0), inb, gsem).wait()

            @pl.when(i > 0)
            def _():
                pltpu.make_async_copy(outb, dst(0), osem).wait()

            @plsc.parallel_loop(0, CHUNK, 1, unroll=UNROLL)
            def _(r):
                _normalize_row(inb, outb, r)

            if not last:
                @pl.when(c + 2 < n_chunks)
                def _():
                    pltpu.async_copy(src(c + 2), inb, gsem)

            pltpu.async_copy(outb, dst(c), osem)

        pltpu.async_copy(src(0), in0, gsem0)
        pltpu.async_copy(src(1), in1, gsem1)

        def pair_body(i, _):
            for b, (inb, outb, gsem, osem) in enumerate(bufs):
                do_chunk(i, 2 * i + b, inb, outb, gsem, osem, last=False)
            return 0

        lax.fori_loop(0, n_chunks // 2, pair_body, 0)
        do_chunk(jnp.int32(n_chunks // 2), n_chunks - 1, in0, out0, gsem0,
                 osem0, last=True)

        pltpu.make_async_copy(out0, dst(0), osem0).wait()
        pltpu.make_async_copy(out1, dst(0), osem1).wait()

    return ka(table)


@functools.partial(jax.jit, static_argnames=("n_rows",))
def _sc_gather_bounce(ids2d, normed, n_rows):
    nw, nc = _num_workers()
    rows_per_w = n_rows // nw
    n_chunks = rows_per_w // CHUNK
    assert rows_per_w % CHUNK == 0 and n_chunks % RING == 0

    @functools.partial(
        pl.kernel,
        out_type=jax.ShapeDtypeStruct((n_rows, HIDDEN), jnp.float32),
        mesh=_make_mesh(),
        scratch_types=[
            pltpu.VMEM((n_chunks, CHUNK), jnp.int32),
        ] + [pltpu.VMEM((CHUNK, HIDDEN), jnp.float32)] * RING
          + [pltpu.SemaphoreType.DMA] * (2 * RING),
    )
    def kb(ids_hbm, normed_hbm, out_hbm, idx_v, *bufs_and_sems):
        bufs = bufs_and_sems[:RING]
        gsems = bufs_and_sems[RING:2 * RING]
        osems = bufs_and_sems[2 * RING:]
        wid = lax.axis_index("s") * nc + lax.axis_index("c")
        w_base = wid * rows_per_w

        pltpu.sync_copy(ids_hbm.at[pl.ds(wid * n_chunks, n_chunks)], idx_v)

        for b in range(RING):
            pltpu.async_copy(normed_hbm.at[idx_v.at[b]], bufs[b], gsems[b])

        def ring_body(i, _):
            for b in range(RING):
                c = RING * i + b
                buf, gsem, osem = bufs[b], gsems[b], osems[b]
                pltpu.make_async_copy(
                    normed_hbm.at[idx_v.at[0]], buf, gsem).wait()
                pltpu.async_copy(
                    buf, out_hbm.at[pl.ds(w_base + c * CHUNK, CHUNK)], osem)

                @pl.when(c + RING < n_chunks)
                def _():
                    pltpu.make_async_copy(
                        buf, out_hbm.at[pl.ds(w_base, CHUNK)], osem).wait()
                    pltpu.async_copy(
                        normed_hbm.at[idx_v.at[c + RING]], buf, gsem)
            return 0

        lax.fori_loop(0, n_chunks // RING, ring_body, 0)

        for b in range(RING):
            pltpu.make_async_copy(
                bufs[b], out_hbm.at[pl.ds(w_base, CHUNK)], osems[b]).wait()

    return kb(ids2d, normed)


def kernel(input_ids, table):
    b, l = input_ids.shape
    n_rows = b * l
    vocab = table.shape[0]
    normed = _sc_normalize_table(table, vocab)
    ids2d = input_ids.reshape(n_rows // CHUNK, CHUNK)
    out = _sc_gather_bounce(ids2d, normed, n_rows)
    return out.reshape(b, l, HIDDEN)
